# Initial kernel scaffold; baseline (speedup 1.0000x reference)
import jax
import jax.numpy as jnp
from jax.experimental import pallas as pl


def kernel(x, edge_index, W_l0, b_l0, W_r0, W_l1, b_l1, W_r1):
    raise NotImplementedError("write your pallas kernel here")


# speedup vs baseline: 4.0780x; 4.0780x over previous
import functools

import jax
import jax.numpy as jnp
from jax import lax
from jax.experimental import pallas as pl
from jax.experimental.pallas import tpu as pltpu
from jax.experimental.pallas import tpu_sc as plsc

D = 128
DP = 128
NC = 2
NS = 16
NW = NC * NS
CHUNK = 128
N_PAD = 10240
RPT = N_PAD // NS


def _sc_aggregate(n_pad, e_pad):
    ept = e_pad // NW
    nch = ept // CHUNK
    mesh = plsc.VectorSubcoreMesh(core_axis_name="c", subcore_axis_name="s")

    def body(x_hbm, src_hbm, dst_hbm, out_hbm, acc_sh, srcv, dstv, idxv,
             rows, sem):
        c = lax.axis_index("c")
        s = lax.axis_index("s")

        zero16 = jnp.zeros((16,), jnp.float32)

        @pl.loop(0, CHUNK)
        def _(i):
            @pl.loop(0, DP, step=16)
            def _(j):
                rows[i, pl.ds(j, 16)] = zero16

        @pl.loop(0, RPT, step=CHUNK)
        def _(r):
            @pl.loop(0, CHUNK, step=16)
            def _(tt):
                idxv[pl.ds(tt, 16)] = lax.iota(jnp.int32, 16) + (
                    s * RPT + r + tt)

            pltpu.sync_copy(rows, acc_sh.at[idxv])

        plsc.subcore_barrier()

        base_ch = (c * NS + s) * nch

        @pl.loop(0, nch)
        def _(e):
            pltpu.sync_copy(src_hbm.at[base_ch + e], srcv)
            pltpu.sync_copy(dst_hbm.at[base_ch + e], dstv)
            pltpu.async_copy(x_hbm.at[srcv], rows, sem).wait()
            pltpu.sync_copy(rows, acc_sh.at[dstv], add=True)

        plsc.subcore_barrier()

        @pl.loop(0, RPT, step=CHUNK)
        def _(r):
            @pl.loop(0, CHUNK, step=16)
            def _(tt):
                idxv[pl.ds(tt, 16)] = lax.iota(jnp.int32, 16) + (
                    s * RPT + r + tt)

            pltpu.sync_copy(acc_sh.at[idxv], rows)
            pltpu.sync_copy(rows, out_hbm.at[c, pl.ds(s * RPT + r, CHUNK)])

    return pl.kernel(
        body,
        out_type=jax.ShapeDtypeStruct((NC, n_pad, DP), jnp.float32),
        mesh=mesh,
        scratch_types=(
            pltpu.VMEM_SHARED((n_pad, DP), jnp.float32),
            pltpu.VMEM((CHUNK,), jnp.int32),
            pltpu.VMEM((CHUNK,), jnp.int32),
            pltpu.VMEM((CHUNK,), jnp.int32),
            pltpu.VMEM((CHUNK, DP), jnp.float32),
            pltpu.SemaphoreType.DMA,
        ),
    )


def _sc_count(n_pad, e_pad):
    ept = e_pad // NW
    nch = ept // CHUNK
    mesh = plsc.VectorSubcoreMesh(core_axis_name="c", subcore_axis_name="s")

    def body(dst_hbm, out_hbm, acc_sh, dstv, idxv, rows):
        c = lax.axis_index("c")
        s = lax.axis_index("s")

        zero16 = jnp.zeros((16,), jnp.float32)

        @pl.loop(0, CHUNK)
        def _(i):
            @pl.loop(0, DP, step=16)
            def _(j):
                rows[i, pl.ds(j, 16)] = zero16

        @pl.loop(0, RPT, step=CHUNK)
        def _(r):
            @pl.loop(0, CHUNK, step=16)
            def _(tt):
                idxv[pl.ds(tt, 16)] = lax.iota(jnp.int32, 16) + (
                    s * RPT + r + tt)

            pltpu.sync_copy(rows, acc_sh.at[idxv])

        one16 = jnp.ones((16,), jnp.float32)

        @pl.loop(0, CHUNK)
        def _(i):
            @pl.loop(0, DP, step=16)
            def _(j):
                rows[i, pl.ds(j, 16)] = one16

        plsc.subcore_barrier()

        base_ch = (c * NS + s) * nch

        @pl.loop(0, nch)
        def _(e):
            pltpu.sync_copy(dst_hbm.at[base_ch + e], dstv)
            pltpu.sync_copy(rows, acc_sh.at[dstv], add=True)

        plsc.subcore_barrier()

        @pl.loop(0, RPT, step=CHUNK)
        def _(r):
            @pl.loop(0, CHUNK, step=16)
            def _(tt):
                idxv[pl.ds(tt, 16)] = lax.iota(jnp.int32, 16) + (
                    s * RPT + r + tt)

            pltpu.sync_copy(acc_sh.at[idxv], rows)
            pltpu.sync_copy(rows, out_hbm.at[c, pl.ds(s * RPT + r, CHUNK)])

    return pl.kernel(
        body,
        out_type=jax.ShapeDtypeStruct((NC, n_pad, DP), jnp.float32),
        mesh=mesh,
        scratch_types=(
            pltpu.VMEM_SHARED((n_pad, DP), jnp.float32),
            pltpu.VMEM((CHUNK,), jnp.int32),
            pltpu.VMEM((CHUNK,), jnp.int32),
            pltpu.VMEM((CHUNK, DP), jnp.float32),
        ),
    )


def _tc_combine(parts, cnt_in, x_in, w_l, b_l, w_r, relu, pad_out):
    n = x_in.shape[0]
    blk = 2000
    assert n % blk == 0
    dout = DP if pad_out else D

    def body(parts_ref, cnt_ref, x_ref, wl_ref, bl_ref, wr_ref,
             out_ref, cnt_out_ref):
        p = parts_ref[...]
        agg = p[0, :, :D] + p[1, :, :D]
        cn = cnt_ref[...]
        if cn.ndim == 3:
            cnt = cn[0, :, 0:1] + cn[1, :, 0:1]
        else:
            cnt = cn
        cnt_out_ref[...] = cnt
        mean = agg / jnp.maximum(cnt, 1.0)
        y = (lax.dot(mean, wl_ref[...], precision=lax.Precision.HIGHEST,
                     preferred_element_type=jnp.float32)
             + bl_ref[...]
             + lax.dot(x_ref[...][:, :D], wr_ref[...],
                       precision=lax.Precision.HIGHEST,
                       preferred_element_type=jnp.float32))
        if relu:
            y = jnp.maximum(y, 0.0)
        out_ref[...] = y

    if cnt_in.ndim == 3:
        cnt_spec = pl.BlockSpec((NC, blk, DP), lambda i: (0, i, 0))
    else:
        cnt_spec = pl.BlockSpec((blk, 1), lambda i: (i, 0))
    specs = [
        pl.BlockSpec((NC, blk, DP), lambda i: (0, i, 0)),
        cnt_spec,
        pl.BlockSpec((blk, DP), lambda i: (i, 0)),
        pl.BlockSpec((D, D), lambda i: (0, 0)),
        pl.BlockSpec((1, D), lambda i: (0, 0)),
        pl.BlockSpec((D, D), lambda i: (0, 0)),
    ]
    args = [parts, cnt_in, x_in, w_l, b_l.reshape(1, D), w_r]
    fn = body
    return pl.pallas_call(
        fn,
        grid=(n // blk,),
        in_specs=specs,
        out_specs=(pl.BlockSpec((blk, dout), lambda i: (i, 0)),
                   pl.BlockSpec((blk, 1), lambda i: (i, 0))),
        out_shape=(jax.ShapeDtypeStruct((n, dout), jnp.float32),
                   jax.ShapeDtypeStruct((n, 1), jnp.float32)),
    )(*args)


def kernel(x, edge_index, W_l0, b_l0, W_r0, W_l1, b_l1, W_r1):
    n = x.shape[0]
    e = edge_index.shape[1]
    e_pad = -(-e // (NW * CHUNK)) * (NW * CHUNK)
    src = edge_index[0]
    dst = edge_index[1]
    pad = e_pad - e
    if pad:
        src = jnp.concatenate([src, jnp.zeros((pad,), jnp.int32)])
        dst = jnp.concatenate([dst, jnp.full((pad,), n, jnp.int32)])
    src = src.reshape(e_pad // CHUNK, CHUNK)
    dst = dst.reshape(e_pad // CHUNK, CHUNK)

    parts0 = _sc_aggregate(N_PAD, e_pad)(x, src, dst)
    cnts = _sc_count(N_PAD, e_pad)(dst)
    h, cnt = _tc_combine(parts0, cnts, x, W_l0, b_l0, W_r0,
                         relu=True, pad_out=False)
    parts1 = _sc_aggregate(N_PAD, e_pad)(h, src, dst)
    out, _ = _tc_combine(parts1, cnt, h, W_l1, b_l1, W_r1,
                         relu=False, pad_out=False)
    return out
